# baseline (speedup 1.0000x reference)
import jax
import jax.numpy as jnp
from jax import lax
from jax.experimental import pallas as pl
from jax.experimental.pallas import tpu as pltpu
from jax.experimental.pallas import tpu_sc as plsc

NC = 2
NS = 16
NW = NC * NS
L = 16

B = 4096
H = 50
D = 32
DW = D // 2

BAGS_PER_W = B // NW
FLAT_PER_W = BAGS_PER_W * H
CHUNK = 128
NCHUNK = FLAT_PER_W // CHUNK
HALF_CHUNKS = NCHUNK // 2
HALF_BAGS = BAGS_PER_W // 2
HPAD = 64
SC_PER_W = BAGS_PER_W * HPAD

_INTER = plsc.PackFormat.INTERLEAVED
_HI = -65536


def _bag_body(idx_hbm, sc_hbm, w_hbm, out_hbm, idx_v, rows_v, sc_v, out_v,
              sem0, sem1):
    wid = lax.axis_index("s") * NC + lax.axis_index("c")

    pltpu.sync_copy(idx_hbm.at[wid], idx_v)
    sc_off = pl.multiple_of(wid * SC_PER_W, SC_PER_W)
    pltpu.sync_copy(sc_hbm.at[pl.ds(sc_off, SC_PER_W)], sc_v)

    copies = []
    for c in range(NCHUNK):
        sem = sem0 if c < HALF_CHUNKS else sem1
        copies.append(
            pltpu.async_copy(w_hbm.at[idx_v.at[c]],
                             rows_v.at[pl.ds(c * CHUNK, CHUNK)], sem))

    def bag(b, carry):
        base = b * H
        sbase = b * HPAD
        s0 = sc_v[pl.ds(sbase, 2 * L)]
        s1 = sc_v[pl.ds(sbase + 2 * L, 2 * L)]
        se0, so0 = plsc.unpack(s0, format=_INTER)
        se1, so1 = plsc.unpack(s1, format=_INTER)
        schunks = (se0, so0, se1, so1)
        acc_e = jnp.zeros((L,), jnp.float32)
        acc_o = jnp.zeros((L,), jnp.float32)
        for l in range(H):
            row = rows_v[base + l, :]
            e = lax.bitcast_convert_type(lax.shift_left(row, 16), jnp.float32)
            o = lax.bitcast_convert_type(
                lax.bitwise_and(row, jnp.full((L,), _HI, jnp.int32)),
                jnp.float32)
            svec = schunks[2 * (l // (2 * L)) + (l & 1)]
            lane = (l % (2 * L)) // 2
            s = jnp.take_along_axis(svec, jnp.full((L,), lane, jnp.int32),
                                    axis=0)
            acc_e = acc_e + s * e
            acc_o = acc_o + s * o
        out_v[b, :] = plsc.pack(acc_e, acc_o, format=_INTER)
        return carry

    for c in range(HALF_CHUNKS):
        copies[c].wait()
    lax.fori_loop(0, HALF_BAGS, bag, 0)
    for c in range(HALF_CHUNKS, NCHUNK):
        copies[c].wait()
    lax.fori_loop(HALF_BAGS, BAGS_PER_W, bag, 0)

    out_off = pl.multiple_of(wid * BAGS_PER_W, BAGS_PER_W)
    pltpu.sync_copy(out_v, out_hbm.at[pl.ds(out_off, BAGS_PER_W)])


@jax.jit
def kernel(indices, scores, weight):
    idx = indices.reshape(NW, NCHUNK, CHUNK)
    sc = jnp.pad(scores, ((0, 0), (0, HPAD - H))).reshape(NW * SC_PER_W)
    w32 = lax.bitcast_convert_type(weight.reshape(weight.shape[0], DW, 2),
                                   jnp.int32)
    run = pl.kernel(
        _bag_body,
        out_type=jax.ShapeDtypeStruct((B, D), jnp.bfloat16),
        mesh=plsc.VectorSubcoreMesh(core_axis_name="c", subcore_axis_name="s"),
        scratch_types=[
            pltpu.VMEM((NCHUNK, CHUNK), jnp.int32),
            pltpu.VMEM((FLAT_PER_W, DW), jnp.int32),
            pltpu.VMEM((SC_PER_W,), jnp.bfloat16),
            pltpu.VMEM((BAGS_PER_W, D), jnp.bfloat16),
            pltpu.SemaphoreType.DMA,
            pltpu.SemaphoreType.DMA,
        ],
        compiler_params=pltpu.CompilerParams(
            needs_layout_passes=False,
            use_tc_tiling_on_sc=False,
        ),
    )
    return run(idx, sc, w32)


# speedup vs baseline: 1.8919x; 1.8919x over previous
import jax
import jax.numpy as jnp
from jax import lax
from jax.experimental import pallas as pl
from jax.experimental.pallas import tpu as pltpu
from jax.experimental.pallas import tpu_sc as plsc

NC = 2
NS = 16
NW = NC * NS
L = 16

B = 4096
H = 50
D = 32
DW = D // 2

BAGS_PER_W = B // NW
FLAT_PER_W = BAGS_PER_W * H
CHUNK = 128
NCHUNK = FLAT_PER_W // CHUNK
HALF_CHUNKS = NCHUNK // 2
HALF_BAGS = BAGS_PER_W // 2
HPAD = 64
SC_PER_W = BAGS_PER_W * HPAD

_INTER = plsc.PackFormat.INTERLEAVED
_HI = -65536


def _bag_body(idx_hbm, sc_hbm, w_hbm, out_hbm, idx_v, rows_v, sc_v, out_v,
              sem0, sem1):
    wid = lax.axis_index("s") * NC + lax.axis_index("c")

    pltpu.sync_copy(idx_hbm.at[wid], idx_v)
    sc_off = pl.multiple_of(wid * SC_PER_W, SC_PER_W)
    pltpu.sync_copy(sc_hbm.at[pl.ds(sc_off, SC_PER_W)], sc_v)

    copies = []
    for c in range(NCHUNK):
        sem = sem0 if c < HALF_CHUNKS else sem1
        copies.append(
            pltpu.async_copy(w_hbm.at[idx_v.at[c]],
                             rows_v.at[pl.ds(c * CHUNK, CHUNK)], sem))

    def bag(b, carry):
        base = b * H
        sbase = b * HPAD
        s0 = sc_v[pl.ds(sbase, 2 * L)]
        s1 = sc_v[pl.ds(sbase + 2 * L, 2 * L)]
        se0, so0 = plsc.unpack(s0, format=_INTER)
        se1, so1 = plsc.unpack(s1, format=_INTER)
        schunks = (se0, so0, se1, so1)
        acc_e = jnp.zeros((L,), jnp.float32)
        acc_o = jnp.zeros((L,), jnp.float32)
        for l in range(H):
            row = rows_v[base + l, :]
            e = lax.bitcast_convert_type(lax.shift_left(row, 16), jnp.float32)
            o = lax.bitcast_convert_type(
                lax.bitwise_and(row, jnp.full((L,), _HI, jnp.int32)),
                jnp.float32)
            svec = schunks[2 * (l // (2 * L)) + (l & 1)]
            lane = (l % (2 * L)) // 2
            s = jnp.take_along_axis(svec, jnp.full((L,), lane, jnp.int32),
                                    axis=0)
            acc_e = acc_e + s * e
            acc_o = acc_o + s * o
        out_v[b, :] = plsc.pack(acc_e, acc_o, format=_INTER)
        return carry

    for c in range(HALF_CHUNKS):
        copies[c].wait()
    lax.fori_loop(0, HALF_BAGS, bag, 0)
    for c in range(HALF_CHUNKS, NCHUNK):
        copies[c].wait()
    lax.fori_loop(HALF_BAGS, BAGS_PER_W, bag, 0)

    out_off = pl.multiple_of(wid * BAGS_PER_W, BAGS_PER_W)
    pltpu.sync_copy(out_v, out_hbm.at[pl.ds(out_off, BAGS_PER_W)])


REPACK_BW = 8192


def _repack_body(wt_ref, out_ref):
    xi = pltpu.bitcast(wt_ref[...], jnp.int32)
    out_ref[...] = xi.T


def _repack(weight):
    size = weight.shape[0]
    wt = weight.T
    grid = pl.cdiv(size, REPACK_BW)
    return pl.pallas_call(
        _repack_body,
        out_shape=jax.ShapeDtypeStruct((size, DW), jnp.int32),
        grid=(grid,),
        in_specs=[pl.BlockSpec((D, REPACK_BW), lambda i: (0, i))],
        out_specs=pl.BlockSpec((REPACK_BW, DW), lambda i: (i, 0)),
    )(wt)


@jax.jit
def kernel(indices, scores, weight):
    idx = indices.reshape(NW, NCHUNK, CHUNK)
    sc = jnp.pad(scores, ((0, 0), (0, HPAD - H))).reshape(NW * SC_PER_W)
    w32 = _repack(weight)
    run = pl.kernel(
        _bag_body,
        out_type=jax.ShapeDtypeStruct((B, D), jnp.bfloat16),
        mesh=plsc.VectorSubcoreMesh(core_axis_name="c", subcore_axis_name="s"),
        scratch_types=[
            pltpu.VMEM((NCHUNK, CHUNK), jnp.int32),
            pltpu.VMEM((FLAT_PER_W, DW), jnp.int32),
            pltpu.VMEM((SC_PER_W,), jnp.bfloat16),
            pltpu.VMEM((BAGS_PER_W, D), jnp.bfloat16),
            pltpu.SemaphoreType.DMA,
            pltpu.SemaphoreType.DMA,
        ],
        compiler_params=pltpu.CompilerParams(
            needs_layout_passes=False,
            use_tc_tiling_on_sc=False,
        ),
    )
    return run(idx, sc, w32)


# speedup vs baseline: 3.1529x; 1.6665x over previous
import jax
import jax.numpy as jnp
from jax import lax
from jax.experimental import pallas as pl
from jax.experimental.pallas import tpu as pltpu
from jax.experimental.pallas import tpu_sc as plsc

NC = 2
NS = 16
NW = NC * NS
L = 16

B = 4096
H = 50
D = 32
DW = D // 2

BAGS_PER_W = B // NW
FLAT_PER_W = BAGS_PER_W * H
CHUNK = 128
NCHUNK = FLAT_PER_W // CHUNK
HALF_CHUNKS = NCHUNK // 2
HALF_BAGS = BAGS_PER_W // 2
HPAD = 64
SC_PER_W = BAGS_PER_W * HPAD

_INTER = plsc.PackFormat.INTERLEAVED
_HI = -65536


def _bag_body(idx_hbm, sc_hbm, w_hbm, out_hbm, idx_v, rows_v, sc_v, out_v,
              sem0, sem1):
    wid = lax.axis_index("s") * NC + lax.axis_index("c")

    pltpu.sync_copy(idx_hbm.at[wid], idx_v)
    sc_off = pl.multiple_of(wid * SC_PER_W, SC_PER_W)
    pltpu.sync_copy(sc_hbm.at[pl.ds(sc_off, SC_PER_W)], sc_v)

    copies = []
    for c in range(NCHUNK):
        sem = sem0 if c < HALF_CHUNKS else sem1
        copies.append(
            pltpu.async_copy(w_hbm.at[idx_v.at[c]],
                             rows_v.at[pl.ds(c * CHUNK, CHUNK)], sem))

    def bag(b, carry):
        base = b * H
        sbase = b * HPAD
        s0 = sc_v[pl.ds(sbase, 2 * L)]
        s1 = sc_v[pl.ds(sbase + 2 * L, 2 * L)]
        se0, so0 = plsc.unpack(s0, format=_INTER)
        se1, so1 = plsc.unpack(s1, format=_INTER)
        schunks = (se0, so0, se1, so1)
        acc_e = jnp.zeros((L,), jnp.float32)
        acc_o = jnp.zeros((L,), jnp.float32)
        for l in range(H):
            row = rows_v[base + l, :]
            e = lax.bitcast_convert_type(lax.shift_left(row, 16), jnp.float32)
            o = lax.bitcast_convert_type(
                lax.bitwise_and(row, jnp.full((L,), _HI, jnp.int32)),
                jnp.float32)
            svec = schunks[2 * (l // (2 * L)) + (l & 1)]
            lane = (l % (2 * L)) // 2
            s = jnp.take_along_axis(svec, jnp.full((L,), lane, jnp.int32),
                                    axis=0)
            acc_e = acc_e + s * e
            acc_o = acc_o + s * o
        out_v[b, :] = plsc.pack(acc_e, acc_o, format=_INTER)
        return carry

    for c in range(HALF_CHUNKS):
        copies[c].wait()
    lax.fori_loop(0, HALF_BAGS, bag, 0)
    for c in range(HALF_CHUNKS, NCHUNK):
        copies[c].wait()
    lax.fori_loop(HALF_BAGS, BAGS_PER_W, bag, 0)

    out_off = pl.multiple_of(wid * BAGS_PER_W, BAGS_PER_W)
    pltpu.sync_copy(out_v, out_hbm.at[pl.ds(out_off, BAGS_PER_W)])


REPACK_BW = 8192


def _repack_body(wt_ref, out_ref, scratch):
    xi = pltpu.bitcast(wt_ref[...], jnp.int32)
    scratch[...] = xi.T
    for m in range(8):
        out_ref[:, 16 * m:16 * m + 16] = scratch[m::8, :]


def _repack(weight):
    size = weight.shape[0]
    wt = weight.T
    grid = pl.cdiv(size, REPACK_BW)
    out = pl.pallas_call(
        _repack_body,
        out_shape=jax.ShapeDtypeStruct((size // 8, 128), jnp.int32),
        grid=(grid,),
        in_specs=[pl.BlockSpec((D, REPACK_BW), lambda i: (0, i))],
        out_specs=pl.BlockSpec((REPACK_BW // 8, 128), lambda i: (i, 0)),
        scratch_shapes=[pltpu.VMEM((REPACK_BW, DW), jnp.int32)],
    )(wt)
    return out.reshape(size, DW)


@jax.jit
def kernel(indices, scores, weight):
    idx = indices.reshape(NW, NCHUNK, CHUNK)
    sc = jnp.pad(scores, ((0, 0), (0, HPAD - H))).reshape(NW * SC_PER_W)
    w32 = _repack(weight)
    run = pl.kernel(
        _bag_body,
        out_type=jax.ShapeDtypeStruct((B, D), jnp.bfloat16),
        mesh=plsc.VectorSubcoreMesh(core_axis_name="c", subcore_axis_name="s"),
        scratch_types=[
            pltpu.VMEM((NCHUNK, CHUNK), jnp.int32),
            pltpu.VMEM((FLAT_PER_W, DW), jnp.int32),
            pltpu.VMEM((SC_PER_W,), jnp.bfloat16),
            pltpu.VMEM((BAGS_PER_W, D), jnp.bfloat16),
            pltpu.SemaphoreType.DMA,
            pltpu.SemaphoreType.DMA,
        ],
        compiler_params=pltpu.CompilerParams(
            needs_layout_passes=False,
            use_tc_tiling_on_sc=False,
        ),
    )
    return run(idx, sc, w32)


# speedup vs baseline: 4.2504x; 1.3481x over previous
import jax
import jax.numpy as jnp
from jax import lax
from jax.experimental import pallas as pl
from jax.experimental.pallas import tpu as pltpu
from jax.experimental.pallas import tpu_sc as plsc

NC = 2
NS = 16
NW = NC * NS
L = 16

B = 4096
H = 50
D = 32
DW = D // 2

BAGS_PER_W = B // NW
FLAT_PER_W = BAGS_PER_W * H
CHUNK = 128
NCHUNK = FLAT_PER_W // CHUNK
HALF_CHUNKS = NCHUNK // 2
HALF_BAGS = BAGS_PER_W // 2
HALF_ROWS = HALF_CHUNKS * CHUNK
HPAD = 64
SC_PER_W = BAGS_PER_W * HPAD

_INTER = plsc.PackFormat.INTERLEAVED
_HI = -65536

REPACK_BW = 8192


def _repack_body(wt_ref, out_ref, scratch):
    x = wt_ref[...]
    ii = lax.broadcasted_iota(jnp.int32, (D, D), 0)
    jj = lax.broadcasted_iota(jnp.int32, (D, D), 1)
    eye = jnp.where(ii == jj, 1.0, 0.0).astype(jnp.bfloat16)
    y = lax.dot_general(x, eye, (((0,), (0,)), ((), ())),
                        preferred_element_type=jnp.float32)
    y = y.astype(jnp.bfloat16)
    z = pltpu.bitcast(y, jnp.int32)
    scratch[...] = z
    for m in range(4):
        out_ref[:, 32 * m:32 * m + 32] = scratch[m::4, :]


def _repack(weight):
    size = weight.shape[0]
    wt = weight.T
    grid = pl.cdiv(size, REPACK_BW)
    out = pl.pallas_call(
        _repack_body,
        out_shape=jax.ShapeDtypeStruct((size // 8, 128), jnp.int32),
        grid=(grid,),
        in_specs=[pl.BlockSpec((D, REPACK_BW), lambda i: (0, i))],
        out_specs=pl.BlockSpec((REPACK_BW // 8, 128), lambda i: (i, 0)),
        scratch_shapes=[pltpu.VMEM((REPACK_BW // 2, D), jnp.int32)],
    )(wt)
    return out.reshape(size // 2, D)


def _bag_body(idx_hbm, par_hbm, sc_hbm, w_hbm, out_hbm,
              idx_v, par_v, rows_v, sc_v, out_v, sem0):
    wid = lax.axis_index("s") * NC + lax.axis_index("c")

    pltpu.sync_copy(idx_hbm.at[wid], idx_v)
    par_off = pl.multiple_of(wid * SC_PER_W, SC_PER_W)
    pltpu.sync_copy(par_hbm.at[pl.ds(par_off, SC_PER_W)], par_v)
    sc_off = pl.multiple_of(wid * SC_PER_W, SC_PER_W)
    pltpu.sync_copy(sc_hbm.at[pl.ds(sc_off, SC_PER_W)], sc_v)

    lanes = lax.iota(jnp.int32, L)
    ilv = (2 * lanes) % L
    half8 = lanes < 8

    def bag(b, carry):
        base = b * H
        roff = (b // HALF_BAGS) * HALF_ROWS
        sbase = b * HPAD
        s0 = sc_v[pl.ds(sbase, 2 * L)]
        s1 = sc_v[pl.ds(sbase + 2 * L, 2 * L)]
        se0, so0 = plsc.unpack(s0, format=_INTER)
        se1, so1 = plsc.unpack(s1, format=_INTER)
        schunks = (se0, so0, se1, so1)
        pgroups = [par_v[pl.ds(sbase + L * g, L)] for g in range(4)]
        acc_lo = jnp.zeros((L,), jnp.float32)
        acc_hi = jnp.zeros((L,), jnp.float32)
        himask = jnp.full((L,), _HI, jnp.int32)
        for l in range(H):
            q = base + l - roff
            w0 = rows_v[q, pl.ds(0, L)]
            w1 = rows_v[q, pl.ds(L, L)]
            samt = jnp.take_along_axis(
                pgroups[l // L], jnp.full((L,), l % L, jnp.int32), axis=0)
            v0 = lax.bitcast_convert_type(
                lax.bitwise_and(lax.shift_left(w0, samt), himask),
                jnp.float32)
            v1 = lax.bitcast_convert_type(
                lax.bitwise_and(lax.shift_left(w1, samt), himask),
                jnp.float32)
            svec = schunks[2 * (l // (2 * L)) + (l & 1)]
            lane = (l % (2 * L)) // 2
            s = jnp.take_along_axis(svec, jnp.full((L,), lane, jnp.int32),
                                    axis=0)
            acc_lo = acc_lo + s * v0
            acc_hi = acc_hi + s * v1
        tl = jnp.take_along_axis(acc_lo, ilv, axis=0)
        th = jnp.take_along_axis(acc_hi, ilv, axis=0)
        e = jnp.where(half8, tl, th)
        tl1 = jnp.take_along_axis(acc_lo, ilv + 1, axis=0)
        th1 = jnp.take_along_axis(acc_hi, ilv + 1, axis=0)
        o = jnp.where(half8, tl1, th1)
        out_v[b, :] = plsc.pack(e, o, format=_INTER)
        return carry

    for half in range(2):
        copies = []
        for c in range(HALF_CHUNKS):
            g = half * HALF_CHUNKS + c
            copies.append(
                pltpu.async_copy(w_hbm.at[idx_v.at[g]],
                                 rows_v.at[pl.ds(c * CHUNK, CHUNK)], sem0))
        for cp in copies:
            cp.wait()
        lax.fori_loop(half * HALF_BAGS, (half + 1) * HALF_BAGS, bag, 0)

    out_off = pl.multiple_of(wid * BAGS_PER_W, BAGS_PER_W)
    pltpu.sync_copy(out_v, out_hbm.at[pl.ds(out_off, BAGS_PER_W)])


@jax.jit
def kernel(indices, scores, weight):
    idx = (indices >> 1).reshape(NW, NCHUNK, CHUNK)
    par = jnp.pad(16 * (1 - (indices & 1)), ((0, 0), (0, HPAD - H)))
    par = par.reshape(NW * SC_PER_W)
    sc = jnp.pad(scores, ((0, 0), (0, HPAD - H))).reshape(NW * SC_PER_W)
    w32 = _repack(weight)
    run = pl.kernel(
        _bag_body,
        out_type=jax.ShapeDtypeStruct((B, D), jnp.bfloat16),
        mesh=plsc.VectorSubcoreMesh(core_axis_name="c", subcore_axis_name="s"),
        scratch_types=[
            pltpu.VMEM((NCHUNK, CHUNK), jnp.int32),
            pltpu.VMEM((SC_PER_W,), jnp.int32),
            pltpu.VMEM((HALF_ROWS, D), jnp.int32),
            pltpu.VMEM((SC_PER_W,), jnp.bfloat16),
            pltpu.VMEM((BAGS_PER_W, D), jnp.bfloat16),
            pltpu.SemaphoreType.DMA,
        ],
        compiler_params=pltpu.CompilerParams(
            needs_layout_passes=False,
            use_tc_tiling_on_sc=False,
        ),
    )
    return run(idx, par, sc, w32)


# speedup vs baseline: 4.2724x; 1.0052x over previous
import jax
import jax.numpy as jnp
from jax import lax
from jax.experimental import pallas as pl
from jax.experimental.pallas import tpu as pltpu
from jax.experimental.pallas import tpu_sc as plsc

NC = 2
NS = 16
NW = NC * NS
L = 16

B = 4096
H = 50
D = 32
DW = D // 2

BAGS_PER_W = B // NW
FLAT_PER_W = BAGS_PER_W * H
CHUNK = 128
NCHUNK = FLAT_PER_W // CHUNK
HALF_CHUNKS = NCHUNK // 2
HALF_BAGS = BAGS_PER_W // 2
HALF_ROWS = HALF_CHUNKS * CHUNK
HPAD = 64
SC_PER_W = BAGS_PER_W * HPAD

_INTER = plsc.PackFormat.INTERLEAVED
_HI = -65536

REPACK_BW = 16384


def _repack_body(wt_ref, out_ref, scratch):
    x = wt_ref[...]
    ii = lax.broadcasted_iota(jnp.int32, (D, D), 0)
    jj = lax.broadcasted_iota(jnp.int32, (D, D), 1)
    eye = jnp.where(ii == jj, 1.0, 0.0).astype(jnp.bfloat16)
    y = lax.dot_general(x, eye, (((0,), (0,)), ((), ())),
                        preferred_element_type=jnp.float32)
    y = y.astype(jnp.bfloat16)
    z = pltpu.bitcast(y, jnp.int32)
    scratch[...] = z
    for m in range(4):
        out_ref[:, 32 * m:32 * m + 32] = scratch[m::4, :]


def _repack(weight):
    size = weight.shape[0]
    wt = weight.T
    grid = pl.cdiv(size, REPACK_BW)
    out = pl.pallas_call(
        _repack_body,
        out_shape=jax.ShapeDtypeStruct((size // 8, 128), jnp.int32),
        grid=(grid,),
        in_specs=[pl.BlockSpec((D, REPACK_BW), lambda i: (0, i))],
        out_specs=pl.BlockSpec((REPACK_BW // 8, 128), lambda i: (i, 0)),
        scratch_shapes=[pltpu.VMEM((REPACK_BW // 2, D), jnp.int32)],
    )(wt)
    return out.reshape(size // 2, D)


def _bag_body(idx_hbm, par_hbm, sc_hbm, w_hbm, out_hbm,
              idx_v, par_v, rows_v, sc_v, out_v, sem0):
    wid = lax.axis_index("s") * NC + lax.axis_index("c")

    pltpu.sync_copy(idx_hbm.at[wid], idx_v)
    par_off = pl.multiple_of(wid * SC_PER_W, SC_PER_W)
    pltpu.sync_copy(par_hbm.at[pl.ds(par_off, SC_PER_W)], par_v)
    sc_off = pl.multiple_of(wid * SC_PER_W, SC_PER_W)
    pltpu.sync_copy(sc_hbm.at[pl.ds(sc_off, SC_PER_W)], sc_v)

    lanes = lax.iota(jnp.int32, L)
    ilv = (2 * lanes) % L
    half8 = lanes < 8

    def bag(b, carry):
        base = b * H
        roff = (b // HALF_BAGS) * HALF_ROWS
        sbase = b * HPAD
        s0 = sc_v[pl.ds(sbase, 2 * L)]
        s1 = sc_v[pl.ds(sbase + 2 * L, 2 * L)]
        se0, so0 = plsc.unpack(s0, format=_INTER)
        se1, so1 = plsc.unpack(s1, format=_INTER)
        schunks = (se0, so0, se1, so1)
        pgroups = [par_v[pl.ds(sbase + L * g, L)] for g in range(4)]
        acc_lo = jnp.zeros((L,), jnp.float32)
        acc_hi = jnp.zeros((L,), jnp.float32)
        himask = jnp.full((L,), _HI, jnp.int32)
        for l in range(H):
            q = base + l - roff
            w0 = rows_v[q, pl.ds(0, L)]
            w1 = rows_v[q, pl.ds(L, L)]
            samt = jnp.take_along_axis(
                pgroups[l // L], jnp.full((L,), l % L, jnp.int32), axis=0)
            v0 = lax.bitcast_convert_type(
                lax.bitwise_and(lax.shift_left(w0, samt), himask),
                jnp.float32)
            v1 = lax.bitcast_convert_type(
                lax.bitwise_and(lax.shift_left(w1, samt), himask),
                jnp.float32)
            svec = schunks[2 * (l // (2 * L)) + (l & 1)]
            lane = (l % (2 * L)) // 2
            s = jnp.take_along_axis(svec, jnp.full((L,), lane, jnp.int32),
                                    axis=0)
            acc_lo = acc_lo + s * v0
            acc_hi = acc_hi + s * v1
        tl = jnp.take_along_axis(acc_lo, ilv, axis=0)
        th = jnp.take_along_axis(acc_hi, ilv, axis=0)
        e = jnp.where(half8, tl, th)
        tl1 = jnp.take_along_axis(acc_lo, ilv + 1, axis=0)
        th1 = jnp.take_along_axis(acc_hi, ilv + 1, axis=0)
        o = jnp.where(half8, tl1, th1)
        out_v[b, :] = plsc.pack(e, o, format=_INTER)
        return carry

    for half in range(2):
        copies = []
        for c in range(HALF_CHUNKS):
            g = half * HALF_CHUNKS + c
            copies.append(
                pltpu.async_copy(w_hbm.at[idx_v.at[g]],
                                 rows_v.at[pl.ds(c * CHUNK, CHUNK)], sem0))
        for cp in copies:
            cp.wait()
        lax.fori_loop(half * HALF_BAGS, (half + 1) * HALF_BAGS, bag, 0)

    out_off = pl.multiple_of(wid * BAGS_PER_W, BAGS_PER_W)
    pltpu.sync_copy(out_v, out_hbm.at[pl.ds(out_off, BAGS_PER_W)])


@jax.jit
def kernel(indices, scores, weight):
    idx = (indices >> 1).reshape(NW, NCHUNK, CHUNK)
    par = jnp.pad(16 * (1 - (indices & 1)), ((0, 0), (0, HPAD - H)))
    par = par.reshape(NW * SC_PER_W)
    sc = jnp.pad(scores, ((0, 0), (0, HPAD - H))).reshape(NW * SC_PER_W)
    w32 = _repack(weight)
    run = pl.kernel(
        _bag_body,
        out_type=jax.ShapeDtypeStruct((B, D), jnp.bfloat16),
        mesh=plsc.VectorSubcoreMesh(core_axis_name="c", subcore_axis_name="s"),
        scratch_types=[
            pltpu.VMEM((NCHUNK, CHUNK), jnp.int32),
            pltpu.VMEM((SC_PER_W,), jnp.int32),
            pltpu.VMEM((HALF_ROWS, D), jnp.int32),
            pltpu.VMEM((SC_PER_W,), jnp.bfloat16),
            pltpu.VMEM((BAGS_PER_W, D), jnp.bfloat16),
            pltpu.SemaphoreType.DMA,
        ],
        compiler_params=pltpu.CompilerParams(
            needs_layout_passes=False,
            use_tc_tiling_on_sc=False,
        ),
    )
    return run(idx, par, sc, w32)


# speedup vs baseline: 4.5714x; 1.0700x over previous
import jax
import jax.numpy as jnp
from jax import lax
from jax.experimental import pallas as pl
from jax.experimental.pallas import tpu as pltpu
from jax.experimental.pallas import tpu_sc as plsc

NC = 2
NS = 16
NW = NC * NS
L = 16

B = 4096
H = 50
D = 32
DW = D // 2

BAGS_PER_W = B // NW
FLAT_PER_W = BAGS_PER_W * H
CHUNK = 128
NCHUNK = FLAT_PER_W // CHUNK
HALF_CHUNKS = NCHUNK // 2
HALF_BAGS = BAGS_PER_W // 2
HALF_ROWS = HALF_CHUNKS * CHUNK
HPAD = 64
SC_PER_W = BAGS_PER_W * HPAD

_INTER = plsc.PackFormat.INTERLEAVED
_HI = -65536

REPACK_BW = 16384


def _repack_body(wt_ref, out_ref, scratch):
    x = wt_ref[...]
    ii = lax.broadcasted_iota(jnp.int32, (D, D), 0)
    jj = lax.broadcasted_iota(jnp.int32, (D, D), 1)
    eye = jnp.where(ii == jj, 1.0, 0.0).astype(jnp.bfloat16)
    y = lax.dot_general(x, eye, (((0,), (0,)), ((), ())),
                        preferred_element_type=jnp.float32)
    y = y.astype(jnp.bfloat16)
    z = pltpu.bitcast(y, jnp.int32)
    scratch[...] = z
    for m in range(4):
        out_ref[:, 32 * m:32 * m + 32] = scratch[m::4, :]


def _repack(weight):
    size = weight.shape[0]
    wt = weight.T
    grid = pl.cdiv(size, REPACK_BW)
    out = pl.pallas_call(
        _repack_body,
        out_shape=jax.ShapeDtypeStruct((size // 8, 128), jnp.int32),
        grid=(grid,),
        in_specs=[pl.BlockSpec((D, REPACK_BW), lambda i: (0, i))],
        out_specs=pl.BlockSpec((REPACK_BW // 8, 128), lambda i: (i, 0)),
        scratch_shapes=[pltpu.VMEM((REPACK_BW // 2, D), jnp.int32)],
    )(wt)
    return out.reshape(size // 2, D)


def _bag_body(idx_hbm, par_hbm, sc_hbm, w_hbm, out_hbm,
              idx_v, par_v, rows_v, sc_v, out_v, sem0):
    wid = lax.axis_index("s") * NC + lax.axis_index("c")

    pltpu.sync_copy(idx_hbm.at[wid], idx_v)
    par_off = pl.multiple_of(wid * SC_PER_W, SC_PER_W)
    pltpu.sync_copy(par_hbm.at[pl.ds(par_off, SC_PER_W)], par_v)
    sc_off = pl.multiple_of(wid * SC_PER_W, SC_PER_W)
    pltpu.sync_copy(sc_hbm.at[pl.ds(sc_off, SC_PER_W)], sc_v)

    lanes = lax.iota(jnp.int32, L)
    ilv = (2 * lanes) % L
    half8 = lanes < 8

    def bag(b, carry):
        base = b * H
        roff = (b // HALF_BAGS) * HALF_ROWS
        sbase = b * HPAD
        s0 = sc_v[pl.ds(sbase, 2 * L)]
        s1 = sc_v[pl.ds(sbase + 2 * L, 2 * L)]
        se0, so0 = plsc.unpack(s0, format=_INTER)
        se1, so1 = plsc.unpack(s1, format=_INTER)
        schunks = (se0, so0, se1, so1)
        pgroups = [par_v[pl.ds(sbase + L * g, L)] for g in range(4)]
        alo = [jnp.zeros((L,), jnp.float32) for _ in range(4)]
        ahi = [jnp.zeros((L,), jnp.float32) for _ in range(4)]
        himask = jnp.full((L,), _HI, jnp.int32)
        for l in range(H):
            q = base + l - roff
            w0 = rows_v[q, pl.ds(0, L)]
            w1 = rows_v[q, pl.ds(L, L)]
            samt = jnp.take_along_axis(
                pgroups[l // L], jnp.full((L,), l % L, jnp.int32), axis=0)
            v0 = lax.bitcast_convert_type(
                lax.bitwise_and(lax.shift_left(w0, samt), himask),
                jnp.float32)
            v1 = lax.bitcast_convert_type(
                lax.bitwise_and(lax.shift_left(w1, samt), himask),
                jnp.float32)
            svec = schunks[2 * (l // (2 * L)) + (l & 1)]
            lane = (l % (2 * L)) // 2
            s = jnp.take_along_axis(svec, jnp.full((L,), lane, jnp.int32),
                                    axis=0)
            alo[l % 4] = alo[l % 4] + s * v0
            ahi[l % 4] = ahi[l % 4] + s * v1
        acc_lo = (alo[0] + alo[1]) + (alo[2] + alo[3])
        acc_hi = (ahi[0] + ahi[1]) + (ahi[2] + ahi[3])
        tl = jnp.take_along_axis(acc_lo, ilv, axis=0)
        th = jnp.take_along_axis(acc_hi, ilv, axis=0)
        e = jnp.where(half8, tl, th)
        tl1 = jnp.take_along_axis(acc_lo, ilv + 1, axis=0)
        th1 = jnp.take_along_axis(acc_hi, ilv + 1, axis=0)
        o = jnp.where(half8, tl1, th1)
        out_v[b, :] = plsc.pack(e, o, format=_INTER)
        return carry

    for half in range(2):
        copies = []
        for c in range(HALF_CHUNKS):
            g = half * HALF_CHUNKS + c
            copies.append(
                pltpu.async_copy(w_hbm.at[idx_v.at[g]],
                                 rows_v.at[pl.ds(c * CHUNK, CHUNK)], sem0))
        for cp in copies:
            cp.wait()
        lax.fori_loop(half * HALF_BAGS, (half + 1) * HALF_BAGS, bag, 0)

    out_off = pl.multiple_of(wid * BAGS_PER_W, BAGS_PER_W)
    pltpu.sync_copy(out_v, out_hbm.at[pl.ds(out_off, BAGS_PER_W)])


@jax.jit
def kernel(indices, scores, weight):
    idx = (indices >> 1).reshape(NW, NCHUNK, CHUNK)
    par = jnp.pad(16 * (1 - (indices & 1)), ((0, 0), (0, HPAD - H)))
    par = par.reshape(NW * SC_PER_W)
    sc = jnp.pad(scores, ((0, 0), (0, HPAD - H))).reshape(NW * SC_PER_W)
    w32 = _repack(weight)
    run = pl.kernel(
        _bag_body,
        out_type=jax.ShapeDtypeStruct((B, D), jnp.bfloat16),
        mesh=plsc.VectorSubcoreMesh(core_axis_name="c", subcore_axis_name="s"),
        scratch_types=[
            pltpu.VMEM((NCHUNK, CHUNK), jnp.int32),
            pltpu.VMEM((SC_PER_W,), jnp.int32),
            pltpu.VMEM((HALF_ROWS, D), jnp.int32),
            pltpu.VMEM((SC_PER_W,), jnp.bfloat16),
            pltpu.VMEM((BAGS_PER_W, D), jnp.bfloat16),
            pltpu.SemaphoreType.DMA,
        ],
        compiler_params=pltpu.CompilerParams(
            needs_layout_passes=False,
            use_tc_tiling_on_sc=False,
        ),
    )
    return run(idx, par, sc, w32)


# speedup vs baseline: 4.7426x; 1.0375x over previous
import jax
import jax.numpy as jnp
from jax import lax
from jax.experimental import pallas as pl
from jax.experimental.pallas import tpu as pltpu
from jax.experimental.pallas import tpu_sc as plsc

NC = 2
NS = 16
NW = NC * NS
L = 16

B = 4096
H = 50
D = 32
DW = D // 2

BAGS_PER_W = B // NW
FLAT_PER_W = BAGS_PER_W * H
CHUNK = 128
NCHUNK = FLAT_PER_W // CHUNK
HALF_CHUNKS = NCHUNK // 2
HALF_BAGS = BAGS_PER_W // 2
HALF_ROWS = HALF_CHUNKS * CHUNK
HPAD = 64
SC_PER_W = BAGS_PER_W * HPAD

_INTER = plsc.PackFormat.INTERLEAVED
_HI = -65536

REPACK_BW = 32768


def _repack_body(wt_ref, out_ref, scratch):
    x = wt_ref[...]
    ii = lax.broadcasted_iota(jnp.int32, (D, D), 0)
    jj = lax.broadcasted_iota(jnp.int32, (D, D), 1)
    eye = jnp.where(ii == jj, 1.0, 0.0).astype(jnp.bfloat16)
    y = lax.dot_general(x, eye, (((0,), (0,)), ((), ())),
                        preferred_element_type=jnp.float32)
    y = y.astype(jnp.bfloat16)
    z = pltpu.bitcast(y, jnp.int32)
    scratch[...] = z
    for m in range(4):
        out_ref[:, 32 * m:32 * m + 32] = scratch[m::4, :]


def _repack(weight):
    size = weight.shape[0]
    wt = weight.T
    grid = pl.cdiv(size, REPACK_BW)
    out = pl.pallas_call(
        _repack_body,
        out_shape=jax.ShapeDtypeStruct((size // 8, 128), jnp.int32),
        grid=(grid,),
        in_specs=[pl.BlockSpec((D, REPACK_BW), lambda i: (0, i))],
        out_specs=pl.BlockSpec((REPACK_BW // 8, 128), lambda i: (i, 0)),
        scratch_shapes=[pltpu.VMEM((REPACK_BW // 2, D), jnp.int32)],
    )(wt)
    return out.reshape(size // 2, D)


def _bag_body(idx_hbm, par_hbm, sc_hbm, w_hbm, out_hbm,
              idx_v, par_v, rows_v, sc_v, out_v, sem0):
    wid = lax.axis_index("s") * NC + lax.axis_index("c")

    pltpu.sync_copy(idx_hbm.at[wid], idx_v)
    par_off = pl.multiple_of(wid * SC_PER_W, SC_PER_W)
    pltpu.sync_copy(par_hbm.at[pl.ds(par_off, SC_PER_W)], par_v)
    sc_off = pl.multiple_of(wid * SC_PER_W, SC_PER_W)
    pltpu.sync_copy(sc_hbm.at[pl.ds(sc_off, SC_PER_W)], sc_v)

    lanes = lax.iota(jnp.int32, L)
    ilv = (2 * lanes) % L
    half8 = lanes < 8

    def bag(b, carry):
        base = b * H
        roff = (b // HALF_BAGS) * HALF_ROWS
        sbase = b * HPAD
        s0 = sc_v[pl.ds(sbase, 2 * L)]
        s1 = sc_v[pl.ds(sbase + 2 * L, 2 * L)]
        se0, so0 = plsc.unpack(s0, format=_INTER)
        se1, so1 = plsc.unpack(s1, format=_INTER)
        schunks = (se0, so0, se1, so1)
        pgroups = [par_v[pl.ds(sbase + L * g, L)] for g in range(4)]
        alo = [jnp.zeros((L,), jnp.float32) for _ in range(4)]
        ahi = [jnp.zeros((L,), jnp.float32) for _ in range(4)]
        himask = jnp.full((L,), _HI, jnp.int32)
        for l in range(H):
            q = base + l - roff
            w0 = rows_v[q, pl.ds(0, L)]
            w1 = rows_v[q, pl.ds(L, L)]
            samt = jnp.take_along_axis(
                pgroups[l // L], jnp.full((L,), l % L, jnp.int32), axis=0)
            v0 = lax.bitcast_convert_type(
                lax.bitwise_and(lax.shift_left(w0, samt), himask),
                jnp.float32)
            v1 = lax.bitcast_convert_type(
                lax.bitwise_and(lax.shift_left(w1, samt), himask),
                jnp.float32)
            svec = schunks[2 * (l // (2 * L)) + (l & 1)]
            lane = (l % (2 * L)) // 2
            s = jnp.take_along_axis(svec, jnp.full((L,), lane, jnp.int32),
                                    axis=0)
            alo[l % 4] = alo[l % 4] + s * v0
            ahi[l % 4] = ahi[l % 4] + s * v1
        acc_lo = (alo[0] + alo[1]) + (alo[2] + alo[3])
        acc_hi = (ahi[0] + ahi[1]) + (ahi[2] + ahi[3])
        tl = jnp.take_along_axis(acc_lo, ilv, axis=0)
        th = jnp.take_along_axis(acc_hi, ilv, axis=0)
        e = jnp.where(half8, tl, th)
        tl1 = jnp.take_along_axis(acc_lo, ilv + 1, axis=0)
        th1 = jnp.take_along_axis(acc_hi, ilv + 1, axis=0)
        o = jnp.where(half8, tl1, th1)
        out_v[b, :] = plsc.pack(e, o, format=_INTER)
        return carry

    for half in range(2):
        copies = []
        for c in range(HALF_CHUNKS):
            g = half * HALF_CHUNKS + c
            copies.append(
                pltpu.async_copy(w_hbm.at[idx_v.at[g]],
                                 rows_v.at[pl.ds(c * CHUNK, CHUNK)], sem0))
        for cp in copies:
            cp.wait()
        lax.fori_loop(half * HALF_BAGS, (half + 1) * HALF_BAGS, bag, 0)

    out_off = pl.multiple_of(wid * BAGS_PER_W, BAGS_PER_W)
    pltpu.sync_copy(out_v, out_hbm.at[pl.ds(out_off, BAGS_PER_W)])


@jax.jit
def kernel(indices, scores, weight):
    idx = (indices >> 1).reshape(NW, NCHUNK, CHUNK)
    par = jnp.pad(16 * (1 - (indices & 1)), ((0, 0), (0, HPAD - H)))
    par = par.reshape(NW * SC_PER_W)
    sc = jnp.pad(scores, ((0, 0), (0, HPAD - H))).reshape(NW * SC_PER_W)
    w32 = _repack(weight)
    run = pl.kernel(
        _bag_body,
        out_type=jax.ShapeDtypeStruct((B, D), jnp.bfloat16),
        mesh=plsc.VectorSubcoreMesh(core_axis_name="c", subcore_axis_name="s"),
        scratch_types=[
            pltpu.VMEM((NCHUNK, CHUNK), jnp.int32),
            pltpu.VMEM((SC_PER_W,), jnp.int32),
            pltpu.VMEM((HALF_ROWS, D), jnp.int32),
            pltpu.VMEM((SC_PER_W,), jnp.bfloat16),
            pltpu.VMEM((BAGS_PER_W, D), jnp.bfloat16),
            pltpu.SemaphoreType.DMA,
        ],
        compiler_params=pltpu.CompilerParams(
            needs_layout_passes=False,
            use_tc_tiling_on_sc=False,
        ),
    )
    return run(idx, par, sc, w32)
